# baseline (speedup 1.0000x reference)
import functools

import jax
import jax.numpy as jnp
from jax import lax
from jax.experimental import pallas as pl
from jax.experimental.pallas import tpu as pltpu
from jax.experimental.pallas import tpu_sc as plsc

_ROWS = 16384
_S = 50
_D = 32
_NC, _NS = 2, 16
_NW = _NC * _NS
_K = 10

_mesh = plsc.VectorSubcoreMesh(core_axis_name="c", subcore_axis_name="s")


def _make_lookup(nj):
    b = nj * _ROWS
    ir = b // 128 // _NW
    g = ir // _K
    pairs = g // 2

    @functools.partial(
        pl.kernel,
        mesh=_mesh,
        compiler_params=pltpu.CompilerParams(use_tc_tiling_on_sc=False),
        out_type=jax.ShapeDtypeStruct((b // 128, 128, _D), jnp.float32),
        scratch_types=[
            pltpu.VMEM((ir, 128), jnp.int32),
            pltpu.VMEM((_K, 128, _D), jnp.float32),
            pltpu.VMEM((_K, 128, _D), jnp.float32),
            pltpu.SemaphoreType.DMA,
            pltpu.SemaphoreType.DMA,
            pltpu.SemaphoreType.DMA,
            pltpu.SemaphoreType.DMA,
        ],
    )
    def _embed_lookup(x_hbm, table_hbm, out_hbm, flat_v, rows0, rows1,
                      g0, g1, s0, s1):
        wid = lax.axis_index("s") * _NC + lax.axis_index("c")
        base = wid * ir

        pltpu.sync_copy(x_hbm.at[pl.ds(base, ir)], flat_v)

        def gather_k(rows, sem, chunk, k):
            return pltpu.make_async_copy(
                table_hbm.at[flat_v.at[chunk * _K + k]],
                rows.at[k],
                sem,
            )

        class _Chunk:
            def __init__(self, rows, sem, chunk):
                self.rows, self.sem, self.chunk = rows, sem, chunk

            def start(self):
                for k in range(_K):
                    gather_k(self.rows, self.sem, self.chunk, k).start()

            def wait(self):
                for k in range(_K):
                    gather_k(self.rows, self.sem, self.chunk, k).wait()

        def gather(rows, sem, chunk):
            return _Chunk(rows, sem, chunk)

        def store(rows, sem, chunk):
            return pltpu.make_async_copy(
                rows,
                out_hbm.at[pl.ds(base + chunk * _K, _K)],
                sem,
            )

        gather(rows0, g0, 0).start()

        def body(t, carry):
            a = 2 * t
            bb = 2 * t + 1

            @pl.when(t > 0)
            def _():
                store(rows1, s1, bb - 2).wait()
            gather(rows1, g1, bb).start()

            gather(rows0, g0, a).wait()
            store(rows0, s0, a).start()

            @pl.when(t < pairs - 1)
            def _():
                store(rows0, s0, a).wait()
                gather(rows0, g0, a + 2).start()

            gather(rows1, g1, bb).wait()
            store(rows1, s1, bb).start()
            return carry

        lax.fori_loop(0, pairs, body, None)
        store(rows0, s0, g - 2).wait()
        store(rows1, s1, g - 1).wait()

    return _embed_lookup


_lookup40 = _make_lookup(40)
_lookup10 = _make_lookup(10)


def kernel(x, table):
    xt = x.T
    halves = []
    for nj, lookup in ((40, _lookup40), (10, _lookup10)):
        j0 = 0 if nj == 40 else 40
        x_flat = xt[j0:j0 + nj].reshape(nj * _ROWS // 128, 128)
        out3 = lookup(x_flat, table)
        halves.append(out3.reshape(nj, _ROWS, _D).transpose(1, 0, 2))
    return jnp.concatenate(halves, axis=1)


# speedup vs baseline: 1.0275x; 1.0275x over previous
import functools

import jax
import jax.numpy as jnp
from jax import lax
from jax.experimental import pallas as pl
from jax.experimental.pallas import tpu as pltpu
from jax.experimental.pallas import tpu_sc as plsc

_ROWS = 16384
_S = 50
_D = 32
_B = _ROWS * _S
_NC, _NS = 2, 16
_NW = _NC * _NS
_BPW = _B // _NW
_IR = _BPW // 128
_K = 10
_C = _K * 128
_G = _BPW // _C
_PAIRS = _G // 2

_mesh = plsc.VectorSubcoreMesh(core_axis_name="c", subcore_axis_name="s")


@functools.partial(
    pl.kernel,
    mesh=_mesh,
    compiler_params=pltpu.CompilerParams(use_tc_tiling_on_sc=False),
    out_type=jax.ShapeDtypeStruct((_B // 128, 128, _D), jnp.float32),
    scratch_types=[
        pltpu.VMEM((_IR, 128), jnp.int32),
        pltpu.VMEM((_K, 128, _D), jnp.float32),
        pltpu.VMEM((_K, 128, _D), jnp.float32),
        pltpu.SemaphoreType.DMA,
        pltpu.SemaphoreType.DMA,
        pltpu.SemaphoreType.DMA,
        pltpu.SemaphoreType.DMA,
    ],
)
def _embed_lookup(x_hbm, table_hbm, out_hbm, flat_v, rows0, rows1,
                  g0, g1, s0, s1):
    wid = lax.axis_index("s") * _NC + lax.axis_index("c")
    base = wid * _IR

    pltpu.sync_copy(x_hbm.at[pl.ds(base, _IR)], flat_v)

    def gather_k(rows, sem, chunk, k):
        return pltpu.make_async_copy(
            table_hbm.at[flat_v.at[chunk * _K + k]],
            rows.at[k],
            sem,
        )

    class _Chunk:
        def __init__(self, rows, sem, chunk):
            self.rows, self.sem, self.chunk = rows, sem, chunk

        def start(self):
            for k in range(_K):
                gather_k(self.rows, self.sem, self.chunk, k).start()

        def wait(self):
            for k in range(_K):
                gather_k(self.rows, self.sem, self.chunk, k).wait()

    def gather(rows, sem, chunk):
        return _Chunk(rows, sem, chunk)

    def store(rows, sem, chunk):
        return pltpu.make_async_copy(
            rows,
            out_hbm.at[pl.ds(base + chunk * _K, _K)],
            sem,
        )

    gather(rows0, g0, 0).start()

    def body(t, carry):
        a = 2 * t
        b = 2 * t + 1

        @pl.when(t > 0)
        def _():
            store(rows1, s1, b - 2).wait()
        gather(rows1, g1, b).start()

        gather(rows0, g0, a).wait()
        store(rows0, s0, a).start()

        @pl.when(t < _PAIRS - 1)
        def _():
            store(rows0, s0, a).wait()
            gather(rows0, g0, a + 2).start()

        gather(rows1, g1, b).wait()
        store(rows1, s1, b).start()
        return carry

    lax.fori_loop(0, _PAIRS, body, None)
    store(rows0, s0, _G - 2).wait()
    store(rows1, s1, _G - 1).wait()


def kernel(x, table):
    x_flat = x.T.reshape(_B // 128, 128)
    out3 = _embed_lookup(x_flat, table)
    return out3.reshape(_S, _ROWS, _D).transpose(1, 0, 2)
